# Initial kernel scaffold; baseline (speedup 1.0000x reference)
import jax
import jax.numpy as jnp
from jax.experimental import pallas as pl


def kernel(query, signatures, query_pos):
    raise NotImplementedError("write your pallas kernel here")


# speedup vs baseline: 1.3582x; 1.3582x over previous
import jax
import jax.numpy as jnp
from jax.experimental import pallas as pl

_NUM_TILES = 64
_LAMBDA = 0.5
_EPS = 1e-12
_BLOCK_M = 1024


def _wm_block_kernel(q_ref, sig_ref, pos_ref, out_ref):
    sig = sig_ref[:]
    sig_inv = 1.0 / jnp.maximum(
        jnp.sqrt(jnp.sum(sig * sig, axis=1)), _EPS)

    q = q_ref[:]
    dot = jax.lax.dot_general(
        q, sig, (((1,), (1,)), ((), ())),
        preferred_element_type=jnp.float32)
    q_inv = 1.0 / jnp.maximum(
        jnp.sqrt(jnp.sum(q * q, axis=1, keepdims=True)), _EPS)
    cos = dot * q_inv * sig_inv[None, :]

    pos = pos_ref[:]
    tiles = jax.lax.broadcasted_iota(
        jnp.int32, (1, _NUM_TILES), 1).astype(jnp.float32)
    d_temporal = jnp.abs(pos - tiles) * (2.0 / (_NUM_TILES - 1))

    out_ref[:] = (1.0 - _LAMBDA) * (1.0 - cos) + _LAMBDA * d_temporal


def kernel(query, signatures, query_pos):
    n, k = query.shape
    pos_f = query_pos.astype(jnp.float32).reshape(n, 1)
    grid = (n // _BLOCK_M,)
    return pl.pallas_call(
        _wm_block_kernel,
        grid=grid,
        in_specs=[
            pl.BlockSpec((_BLOCK_M, k), lambda i: (i, 0)),
            pl.BlockSpec((_NUM_TILES, k), lambda i: (0, 0)),
            pl.BlockSpec((_BLOCK_M, 1), lambda i: (i, 0)),
        ],
        out_specs=pl.BlockSpec((_BLOCK_M, _NUM_TILES), lambda i: (i, 0)),
        out_shape=jax.ShapeDtypeStruct((n, _NUM_TILES), jnp.float32),
    )(query, signatures, pos_f)


# speedup vs baseline: 1.3601x; 1.0014x over previous
import jax
import jax.numpy as jnp
from jax.experimental import pallas as pl
from jax.experimental.pallas import tpu as pltpu

_NUM_TILES = 64
_LAMBDA = 0.5
_EPS = 1e-12
_BLOCK_M = 1024


def _wm_block_kernel(q_ref, sig_ref, pos_ref, out_ref):
    sig = sig_ref[:]
    sig_inv = 1.0 / jnp.maximum(
        jnp.sqrt(jnp.sum(sig * sig, axis=1)), _EPS)

    q = q_ref[:]
    dot = jax.lax.dot_general(
        q, sig, (((1,), (1,)), ((), ())),
        preferred_element_type=jnp.float32)
    q_inv = 1.0 / jnp.maximum(
        jnp.sqrt(jnp.sum(q * q, axis=1, keepdims=True)), _EPS)
    cos = dot * q_inv * sig_inv[None, :]

    pos = pos_ref[:]
    tiles = jax.lax.broadcasted_iota(
        jnp.int32, (1, _NUM_TILES), 1).astype(jnp.float32)
    d_temporal = jnp.abs(pos - tiles) * (2.0 / (_NUM_TILES - 1))

    out_ref[:] = (1.0 - _LAMBDA) * (1.0 - cos) + _LAMBDA * d_temporal


def kernel(query, signatures, query_pos):
    n, k = query.shape
    pos_f = query_pos.astype(jnp.float32).reshape(n, 1)
    grid = (n // _BLOCK_M,)
    return pl.pallas_call(
        _wm_block_kernel,
        grid=grid,
        in_specs=[
            pl.BlockSpec((_BLOCK_M, k), lambda i: (i, 0)),
            pl.BlockSpec((_NUM_TILES, k), lambda i: (0, 0)),
            pl.BlockSpec((_BLOCK_M, 1), lambda i: (i, 0)),
        ],
        out_specs=pl.BlockSpec((_BLOCK_M, _NUM_TILES), lambda i: (i, 0)),
        out_shape=jax.ShapeDtypeStruct((n, _NUM_TILES), jnp.float32),
        compiler_params=pltpu.CompilerParams(
            dimension_semantics=("parallel",)),
    )(query, signatures, pos_f)
